# Initial kernel scaffold; baseline (speedup 1.0000x reference)
import jax
import jax.numpy as jnp
from jax.experimental import pallas as pl


def kernel(inputs, down_c1_ws, down_c1_wn, down_c1_b, down_c2_ws, down_c2_wn, down_c2_b, low_c1_ws, low_c1_wn, low_c1_b, low_c2_ws, low_c2_wn, low_c2_b, up_c1_ws, up_c1_wn, up_c1_b, up_c2_ws, up_c2_wn, up_c2_b):
    raise NotImplementedError("write your pallas kernel here")


# speedup vs baseline: 407.2518x; 407.2518x over previous
import jax
import jax.numpy as jnp
from jax.experimental import pallas as pl


def _nb_mean(x):
    return (jnp.roll(x, 1, 0) + jnp.roll(x, -1, 0)
            + jnp.roll(x, 1, 1) + jnp.roll(x, -1, 1)) * 0.25


def _sage(x, ws, wn, b):
    nx = x.shape[0]
    m = _nb_mean(x)
    y = x.reshape(nx * nx, -1) @ ws + m.reshape(nx * nx, -1) @ wn + b
    return jax.nn.relu(y).reshape(nx, nx, -1)


def _pool(x):
    nx, ny, c = x.shape
    a = x.reshape(nx // 2, 2, ny, c)
    a = a[:, 0] + a[:, 1]
    b = a.reshape(nx // 2, ny // 2, 2, c)
    b = b[:, :, 0] + b[:, :, 1]
    return b * 0.25


def _upsample(x):
    nx, ny, c = x.shape
    u = jnp.broadcast_to(x[:, None], (nx, 2, ny, c)).reshape(2 * nx, ny, c)
    u = jnp.broadcast_to(u[:, :, None], (2 * nx, ny, 2, c)).reshape(2 * nx, 2 * ny, c)
    return u


def _unet_slab(x_ref,
               dc1ws, dc1wn, dc1b, dc2ws, dc2wn, dc2b,
               lc1ws, lc1wn, lc1b, lc2ws, lc2wn, lc2b,
               uc1ws, uc1wn, uc1b, uc2ws, uc2wn, uc2b,
               out_ref):
    x = x_ref[0]
    x = _sage(x, dc1ws[...], dc1wn[...], dc1b[...])
    x = _sage(x, dc2ws[...], dc2wn[...], dc2b[...])
    skip = x
    p = _pool(x)
    p = _sage(p, lc1ws[...], lc1wn[...], lc1b[...])
    p = _sage(p, lc2ws[...], lc2wn[...], lc2b[...])
    u = _upsample(p)
    cat = jnp.concatenate([u, skip], axis=-1)
    y = _sage(cat, uc1ws[...], uc1wn[...], uc1b[...])
    y = _sage(y, uc2ws[...], uc2wn[...], uc2b[...])
    out_ref[0] = y


def kernel(inputs,
           down_c1_ws, down_c1_wn, down_c1_b,
           down_c2_ws, down_c2_wn, down_c2_b,
           low_c1_ws, low_c1_wn, low_c1_b,
           low_c2_ws, low_c2_wn, low_c2_b,
           up_c1_ws, up_c1_wn, up_c1_b,
           up_c2_ws, up_c2_wn, up_c2_b):
    B, T, NX, NY, CIN = inputs.shape
    H = down_c1_ws.shape[1]
    x = inputs.reshape(B * T, NX, NY, CIN)

    weights = (down_c1_ws, down_c1_wn, down_c1_b.reshape(1, -1),
               down_c2_ws, down_c2_wn, down_c2_b.reshape(1, -1),
               low_c1_ws, low_c1_wn, low_c1_b.reshape(1, -1),
               low_c2_ws, low_c2_wn, low_c2_b.reshape(1, -1),
               up_c1_ws, up_c1_wn, up_c1_b.reshape(1, -1),
               up_c2_ws, up_c2_wn, up_c2_b.reshape(1, -1))

    in_specs = [pl.BlockSpec((1, NX, NY, CIN), lambda i: (i, 0, 0, 0))]
    for w in weights:
        in_specs.append(pl.BlockSpec(w.shape, lambda i: (0,) * w.ndim))

    out = pl.pallas_call(
        _unet_slab,
        grid=(B * T,),
        in_specs=in_specs,
        out_specs=pl.BlockSpec((1, NX, NY, H), lambda i: (i, 0, 0, 0)),
        out_shape=jax.ShapeDtypeStruct((B * T, NX, NY, H), jnp.float32),
    )(x, *weights)
    return out.reshape(B, T, NX, NY, H)


# speedup vs baseline: 616.9769x; 1.5150x over previous
import jax
import jax.numpy as jnp
from jax.experimental import pallas as pl
from jax.experimental.pallas import tpu as pltpu


def _nb_mean(x):
    return (jnp.roll(x, 1, 0) + jnp.roll(x, -1, 0)
            + jnp.roll(x, 1, 1) + jnp.roll(x, -1, 1)) * 0.25


def _sage(x, ws, wn, b):
    nx = x.shape[0]
    m = _nb_mean(x)
    y = x.reshape(nx * nx, -1) @ ws + m.reshape(nx * nx, -1) @ wn + b
    return jax.nn.relu(y).reshape(nx, nx, -1)


def _pool(x):
    nx, ny, c = x.shape
    a = x.reshape(nx // 2, 2, ny, c)
    a = a[:, 0] + a[:, 1]
    b = a.reshape(nx // 2, ny // 2, 2, c)
    b = b[:, :, 0] + b[:, :, 1]
    return b * 0.25


def _upsample(x):
    nx, ny, c = x.shape
    u = jnp.broadcast_to(x[:, None], (nx, 2, ny, c)).reshape(2 * nx, ny, c)
    u = jnp.broadcast_to(u[:, :, None], (2 * nx, ny, 2, c)).reshape(2 * nx, 2 * ny, c)
    return u


def _unet_pair(x_ref,
               dc1ws, dc1wn, dc1b, dc2ws, dc2wn, dc2b,
               lc1ws, lc1wn, lc1b, lc2ws, lc2wn, lc2b,
               uc1ws_u, uc1ws_s, uc1wn_u, uc1wn_s, uc1b,
               uc2ws, uc2wn, uc2b,
               out_ref):
    h = out_ref.shape[-1]
    x = jnp.concatenate([x_ref[0], x_ref[1]], axis=-1)
    x = _sage(x, dc1ws[...], dc1wn[...], dc1b[...])
    x = _sage(x, dc2ws[...], dc2wn[...], dc2b[...])
    skip = x
    p = _pool(x)
    p = _sage(p, lc1ws[...], lc1wn[...], lc1b[...])
    p = _sage(p, lc2ws[...], lc2wn[...], lc2b[...])
    u = _upsample(p)
    nx = u.shape[0]
    uf = u.reshape(nx * nx, -1)
    sf = skip.reshape(nx * nx, -1)
    hs = uf @ uc1ws_u[...] + sf @ uc1ws_s[...]
    hn = (uf @ uc1wn_u[...] + sf @ uc1wn_s[...]).reshape(nx, nx, -1)
    y = jax.nn.relu(hs.reshape(nx, nx, -1) + _nb_mean(hn) + uc1b[...])
    y = _sage(y, uc2ws[...], uc2wn[...], uc2b[...])
    out_ref[0] = y[..., :h]
    out_ref[1] = y[..., h:]


def _diag2(w):
    ci, co = w.shape
    z = jnp.zeros_like(w)
    return jnp.concatenate(
        [jnp.concatenate([w, z], axis=1), jnp.concatenate([z, w], axis=1)],
        axis=0)


def kernel(inputs,
           down_c1_ws, down_c1_wn, down_c1_b,
           down_c2_ws, down_c2_wn, down_c2_b,
           low_c1_ws, low_c1_wn, low_c1_b,
           low_c2_ws, low_c2_wn, low_c2_b,
           up_c1_ws, up_c1_wn, up_c1_b,
           up_c2_ws, up_c2_wn, up_c2_b):
    B, T, NX, NY, CIN = inputs.shape
    H = down_c1_ws.shape[1]
    S = B * T
    G = S // 2
    x = inputs.reshape(S, NX, NY, CIN)

    def b2(b):
        return jnp.concatenate([b, b]).reshape(1, 2 * b.shape[0])

    weights = (
        _diag2(down_c1_ws), _diag2(down_c1_wn), b2(down_c1_b),
        _diag2(down_c2_ws), _diag2(down_c2_wn), b2(down_c2_b),
        _diag2(low_c1_ws), _diag2(low_c1_wn), b2(low_c1_b),
        _diag2(low_c2_ws), _diag2(low_c2_wn), b2(low_c2_b),
        _diag2(up_c1_ws[:H]), _diag2(up_c1_ws[H:]),
        _diag2(up_c1_wn[:H]), _diag2(up_c1_wn[H:]), b2(up_c1_b),
        _diag2(up_c2_ws), _diag2(up_c2_wn), b2(up_c2_b),
    )

    in_specs = [pl.BlockSpec((2, NX, NY, CIN), lambda i: (i, 0, 0, 0))]
    for w in weights:
        in_specs.append(pl.BlockSpec(w.shape, lambda i: (0,) * w.ndim))

    out = pl.pallas_call(
        _unet_pair,
        grid=(G,),
        in_specs=in_specs,
        out_specs=pl.BlockSpec((2, NX, NY, H), lambda i: (i, 0, 0, 0)),
        out_shape=jax.ShapeDtypeStruct((S, NX, NY, H), jnp.float32),
        compiler_params=pltpu.CompilerParams(
            vmem_limit_bytes=64 * 1024 * 1024),
    )(x, *weights)
    return out.reshape(B, T, NX, NY, H)
